# Initial kernel scaffold; baseline (speedup 1.0000x reference)
import functools

import jax
import jax.numpy as jnp
from jax import lax
from jax.experimental import pallas as pl
from jax.experimental.pallas import tpu as pltpu
from jax.experimental.pallas import tpu_sc as plsc

N = 10000
E = 160000
NPRED = 100000
NC = 2
NS = 16
NT = NC * NS

EB = 128
EPT = 10240
E_PAD = EPT * NS
NBLK = EPT // EB
ROWS_PT = N // NS
ACC_ROWS = N + NS
PAD_PRED = 102400
GROWS = 2 * PAD_PRED
GPT = GROWS // NT
DBLK = GPT // EB

_mesh = plsc.VectorSubcoreMesh(
    core_axis_name="c", subcore_axis_name="s", num_cores=NC, num_subcores=NS)


def _deg_body(dst_hbm, out_hbm, acc, zbuf, ones_rows, idxd, ssem):
    cid = lax.axis_index("c")
    sid = lax.axis_index("s")

    @pl.when(cid == 0)
    def _():
        zero16 = jnp.zeros((16,), jnp.float32)
        one16 = jnp.ones((16,), jnp.float32)

        def fill(i, _):
            zbuf[i, :] = zero16
            return 0
        lax.fori_loop(0, ROWS_PT, fill, 0)

        def fill2(i, _):
            ones_rows[i, :] = one16
            return 0
        lax.fori_loop(0, EB, fill2, 0)

        pltpu.sync_copy(zbuf, acc.at[pl.ds(sid * ROWS_PT, ROWS_PT)])
        plsc.subcore_barrier()

        ebase = sid * EPT

        def blk(j, _):
            b = lax.rem(j, 2)
            pltpu.sync_copy(dst_hbm.at[pl.ds(ebase + j * EB, EB)], idxd.at[b])
            pltpu.async_copy(ones_rows, acc.at[idxd.at[b]], ssem, add=True
                             ).wait()
            return 0
        lax.fori_loop(0, NBLK, blk, 0)
        plsc.subcore_barrier()
        pltpu.sync_copy(acc.at[pl.ds(sid * ROWS_PT, ROWS_PT)],
                        out_hbm.at[pl.ds(sid * ROWS_PT, ROWS_PT)])


_deg_call = functools.partial(
    pl.kernel,
    out_type=jax.ShapeDtypeStruct((N, 16), jnp.float32),
    mesh=_mesh,
    scratch_types=[
        pltpu.VMEM_SHARED((ACC_ROWS, 16), jnp.float32),
        pltpu.VMEM((ROWS_PT, 16), jnp.float32),
        pltpu.VMEM((EB, 16), jnp.float32),
        pltpu.VMEM((2, EB), jnp.int32),
        pltpu.SemaphoreType.DMA,
    ],
)(_deg_body)


def _make_agg(K):
    GB = 4

    def body(h, sidx, dst, out, acc, idxs, idxd, rows, gsem, ssem):
        cid = lax.axis_index("c")
        sid = lax.axis_index("s")
        ebase = sid * EPT
        for p in range(K // NC):
            c = cid * (K // NC) + p
            crow = c * N
            pltpu.sync_copy(h.at[pl.ds(crow + sid * ROWS_PT, ROWS_PT)],
                            acc.at[pl.ds(sid * ROWS_PT, ROWS_PT)])
            plsc.subcore_barrier()

            def grp(g, _):
                gets = []
                for b in range(GB):
                    off = ebase + (g * GB + b) * EB
                    pltpu.sync_copy(sidx.at[c, pl.ds(off, EB)], idxs.at[b])
                    pltpu.sync_copy(dst.at[pl.ds(off, EB)], idxd.at[b])
                    gets.append(
                        pltpu.async_copy(h.at[idxs.at[b]], rows.at[b], gsem))
                for hd in gets:
                    hd.wait()
                puts = [
                    pltpu.async_copy(rows.at[b], acc.at[idxd.at[b]], ssem,
                                     add=True)
                    for b in range(GB)
                ]
                for hd in puts:
                    hd.wait()
                return 0
            lax.fori_loop(0, NBLK // GB, grp, 0)
            plsc.subcore_barrier()
            pltpu.sync_copy(acc.at[pl.ds(sid * ROWS_PT, ROWS_PT)],
                            out.at[pl.ds(crow + sid * ROWS_PT, ROWS_PT)])

    return functools.partial(
        pl.kernel,
        out_type=jax.ShapeDtypeStruct((K * N, 128), jnp.float32),
        mesh=_mesh,
        scratch_types=[
            pltpu.VMEM_SHARED((ACC_ROWS, 128), jnp.float32),
            pltpu.VMEM((GB, EB), jnp.int32),
            pltpu.VMEM((GB, EB), jnp.int32),
            pltpu.VMEM((GB, EB, 128), jnp.float32),
            pltpu.SemaphoreType.DMA,
            pltpu.SemaphoreType.DMA,
        ],
    )(body)


_agg4_call = _make_agg(4)
_agg2_call = _make_agg(2)


def _gather_body(z_hbm, idx_hbm, out_hbm, idxv, rowsv, gsem, osem):
    cid = lax.axis_index("c")
    sid = lax.axis_index("s")
    base = (sid * NC + cid) * GPT

    def grp(g, _):
        gets = []
        for b in range(2):
            off = base + (g * 2 + b) * EB
            pltpu.sync_copy(idx_hbm.at[pl.ds(off, EB)], idxv.at[b])
            gets.append(pltpu.async_copy(z_hbm.at[idxv.at[b]], rowsv.at[b],
                                         gsem))
        for hd in gets:
            hd.wait()
        puts = [
            pltpu.async_copy(rowsv.at[b],
                             out_hbm.at[pl.ds(base + (g * 2 + b) * EB, EB)],
                             osem)
            for b in range(2)
        ]
        for hd in puts:
            hd.wait()
        return 0
    lax.fori_loop(0, DBLK // 2, grp, 0)


_gather_call = functools.partial(
    pl.kernel,
    out_type=jax.ShapeDtypeStruct((GROWS, 256), jnp.float32),
    mesh=_mesh,
    scratch_types=[
        pltpu.VMEM((2, EB), jnp.int32),
        pltpu.VMEM((2, EB, 256), jnp.float32),
        pltpu.SemaphoreType.DMA,
        pltpu.SemaphoreType.DMA,
    ],
)(_gather_body)


def _mm_t(x, w):
    return lax.dot_general(x, w, (((1,), (1,)), ((), ())),
                           preferred_element_type=jnp.float32)


def _lrelu(x):
    return jnp.where(x > 0, x, x * 0.01)


def _k1_body(feat, pe31, deg, wpos, bpos, w1, b1, w2, b2, w3, b3, wc1, out):
    x = feat[0]
    pos = _mm_t(pe31[0], wpos[...]) + bpos[...]
    a = _mm_t(x, w1[0, :, :256]) + _mm_t(pos, w1[0, :, 256:]) + b1[0]
    a = _lrelu(a)
    a = _lrelu(_mm_t(a, w2[0]) + b2[0])
    a = _lrelu(_mm_t(a, w3[0]) + b3[0])
    h = _mm_t(a, wc1[...])
    dinv = lax.rsqrt(deg[0][:, 0:1] + 1.0)
    h = h * dinv
    for k in range(4):
        out[k, 0] = h[:, 128 * k:128 * (k + 1)]


def _k1_call(featr, pe31, degr, wpos, bpos, w1, b1, w2, b2, w3, b3, wc1):
    f32 = jnp.float32
    return pl.pallas_call(
        _k1_body,
        grid=(8,),
        in_specs=[
            pl.BlockSpec((1, 1250, 256), lambda i: (i, 0, 0)),
            pl.BlockSpec((1, 1250, 31), lambda i: (i, 0, 0)),
            pl.BlockSpec((1, 1250, 16), lambda i: (i, 0, 0)),
            pl.BlockSpec((8, 31), lambda i: (0, 0)),
            pl.BlockSpec((1, 8), lambda i: (0, 0)),
            pl.BlockSpec((1, 264, 264), lambda i: (i, 0, 0)),
            pl.BlockSpec((1, 1, 264), lambda i: (i, 0, 0)),
            pl.BlockSpec((1, 264, 264), lambda i: (i, 0, 0)),
            pl.BlockSpec((1, 1, 264), lambda i: (i, 0, 0)),
            pl.BlockSpec((1, 264, 264), lambda i: (i, 0, 0)),
            pl.BlockSpec((1, 1, 264), lambda i: (i, 0, 0)),
            pl.BlockSpec((512, 264), lambda i: (0, 0)),
        ],
        out_specs=pl.BlockSpec((4, 1, 1250, 128), lambda i: (0, i, 0, 0)),
        out_shape=jax.ShapeDtypeStruct((4, 8, 1250, 128), f32),
    )(featr, pe31, degr, wpos, bpos, w1, b1, w2, b2, w3, b3, wc1)


def _make_gcn_mid(k_in, k_out):
    def body(agg, deg, bias, w, out):
        x = jnp.concatenate([agg[k] for k in range(k_in)], axis=-1)
        dinv = lax.rsqrt(deg[0][:, 0:1] + 1.0)
        x = jnp.maximum(x * dinv + bias[...], 0.0)
        y = _mm_t(x, w[...]) * dinv
        for k in range(k_out):
            out[k] = y[:, 128 * k:128 * (k + 1)]

    def call(aggr, degr, bias, w):
        d_out = 128 * k_out
        return pl.pallas_call(
            body,
            grid=(10,),
            in_specs=[
                pl.BlockSpec((k_in, 1000, 128), lambda j: (0, j, 0)),
                pl.BlockSpec((1, 1000, 16), lambda j: (j, 0, 0)),
                pl.BlockSpec((1, 128 * k_in), lambda j: (0, 0)),
                pl.BlockSpec((d_out, 128 * k_in), lambda j: (0, 0)),
            ],
            out_specs=pl.BlockSpec((k_out, 1000, 128), lambda j: (0, j, 0)),
            out_shape=jax.ShapeDtypeStruct((k_out, N, 128), jnp.float32),
        )(aggr, degr, bias, w)

    return call


_gcn2_call = _make_gcn_mid(4, 4)
_gcn3_call = _make_gcn_mid(4, 2)


def _z_body(agg, deg, bias, out):
    x = jnp.concatenate([agg[0], agg[1]], axis=-1)
    dinv = lax.rsqrt(deg[0][:, 0:1] + 1.0)
    out[...] = x * dinv + bias[...]


def _z_call(aggr, degr, bias):
    return pl.pallas_call(
        _z_body,
        grid=(10,),
        in_specs=[
            pl.BlockSpec((2, 1000, 128), lambda j: (0, j, 0)),
            pl.BlockSpec((1, 1000, 16), lambda j: (j, 0, 0)),
            pl.BlockSpec((1, 256), lambda j: (0, 0)),
        ],
        out_specs=pl.BlockSpec((1000, 256), lambda j: (j, 0)),
        out_shape=jax.ShapeDtypeStruct((N, 256), jnp.float32),
    )(aggr, degr, bias)


def _dec_body(zi, zj, wd1, bd1, wd2, bd2, out):
    zz = zi[...] * zj[...]
    h = jnp.maximum(_mm_t(zz, wd1[...]) + bd1[...], 0.0)
    y = _mm_t(h, wd2[...]) + bd2[...]
    out[...] = 1.0 / (1.0 + jnp.exp(-y))


def _dec_call(zrows, wd1, bd1, wd2, bd2):
    nb = 98
    return pl.pallas_call(
        _dec_body,
        grid=(nb,),
        in_specs=[
            pl.BlockSpec((1024, 256), lambda j: (j, 0)),
            pl.BlockSpec((1024, 256), lambda j: (j + PAD_PRED // 1024, 0)),
            pl.BlockSpec((256, 256), lambda j: (0, 0)),
            pl.BlockSpec((1, 256), lambda j: (0, 0)),
            pl.BlockSpec((1, 256), lambda j: (0, 0)),
            pl.BlockSpec((1, 1), lambda j: (0, 0)),
        ],
        out_specs=pl.BlockSpec((1024, 1), lambda j: (j, 0)),
        out_shape=jax.ShapeDtypeStruct((nb * 1024, 1), jnp.float32),
    )(zrows, zrows, wd1, bd1, wd2, bd2)


def kernel(adj, features, edges, com_xs, pos_emb, lap_pe, Wpos, bpos,
           W1, b1, W2, b2, W3, b3, Wc1, bc1, Wc2, bc2, Wc3, bc3,
           Wd1, bd1, Wd2, bd2):
    i32 = jnp.int32
    src = adj[0].astype(i32)
    dst = adj[1].astype(i32)

    npad = E_PAD - E
    filler = jnp.arange(npad, dtype=i32)
    src_pad = jnp.concatenate([src, filler % 256])
    dst_pad = jnp.concatenate([dst, N + (filler % NS)])
    srcidx = src_pad[None, :] + (jnp.arange(4, dtype=i32) * N)[:, None]

    ppad = PAD_PRED - NPRED
    pfill = jnp.arange(ppad, dtype=i32) % 256
    ei = edges[:, 0].astype(i32)
    ej = edges[:, 1].astype(i32)
    eij = jnp.concatenate([ei, pfill, ej, pfill])

    pe31 = jnp.concatenate([pos_emb, lap_pe], axis=1).reshape(8, 1250, 31)
    featr = features.reshape(8, 1250, 256)

    deg16 = _deg_call(dst_pad)
    degr8 = deg16.reshape(8, 1250, 16)
    degr10 = deg16.reshape(10, 1000, 16)

    h1p = _k1_call(featr, pe31, degr8, Wpos, bpos.reshape(1, 8),
                   W1, b1.reshape(8, 1, 264), W2, b2.reshape(8, 1, 264),
                   W3, b3.reshape(8, 1, 264), Wc1)
    h1p = h1p.reshape(4 * N, 128)

    agg1 = _agg4_call(h1p, srcidx, dst_pad).reshape(4, N, 128)
    h2p = _gcn2_call(agg1, degr10, bc1.reshape(1, 512), Wc2)
    h2p = h2p.reshape(4 * N, 128)

    agg2 = _agg4_call(h2p, srcidx, dst_pad).reshape(4, N, 128)
    h3p = _gcn3_call(agg2, degr10, bc2.reshape(1, 512), Wc3)
    h3p = h3p.reshape(2 * N, 128)

    agg3 = _agg2_call(h3p, srcidx, dst_pad).reshape(2, N, 128)
    z = _z_call(agg3, degr10, bc3.reshape(1, 256))

    zrows = _gather_call(z, eij)
    y = _dec_call(zrows, Wd1, bd1.reshape(1, 256), Wd2.reshape(1, 256),
                  bd2.reshape(1, 1))
    return y[:NPRED, 0]


# speedup vs baseline: 7.6303x; 7.6303x over previous
import functools

import jax
import jax.numpy as jnp
from jax import lax
from jax.experimental import pallas as pl
from jax.experimental.pallas import tpu as pltpu
from jax.experimental.pallas import tpu_sc as plsc

N = 10000
E = 160000
NPRED = 100000
NC = 2
NS = 16
NT = NC * NS

EB = 128
EPT = 10240
E_PAD = EPT * NS
NBLK = EPT // EB
ROWS_PT = 624
ROWS_REM = N - NS * ROWS_PT
ACC_ROWS = N + NS
PAD_PRED = 102400
GROWS = 2 * PAD_PRED
GPT = GROWS // NT
DBLK = GPT // EB

@functools.lru_cache
def _get_mesh():
    return plsc.VectorSubcoreMesh(
        core_axis_name="c", subcore_axis_name="s",
        num_cores=NC, num_subcores=NS)


def _deg_body(dst_hbm, out_hbm, acc, zbuf, ones_rows, idxd, ssem):
    cid = lax.axis_index("c")
    sid = lax.axis_index("s")

    @pl.when(cid == 0)
    def _():
        zero16 = jnp.zeros((16,), jnp.float32)
        one16 = jnp.ones((16,), jnp.float32)

        def fill(i, _):
            zbuf[i, :] = zero16
            return 0
        lax.fori_loop(0, ROWS_PT, fill, 0)

        def fill2(i, _):
            ones_rows[i, :] = one16
            return 0
        lax.fori_loop(0, EB, fill2, 0)

        roff = pl.multiple_of(sid * ROWS_PT, 8)
        pltpu.sync_copy(zbuf, acc.at[pl.ds(roff, ROWS_PT)])

        @pl.when(sid == 0)
        def _rem():
            pltpu.sync_copy(zbuf.at[pl.ds(0, ROWS_REM)],
                            acc.at[pl.ds(NS * ROWS_PT, ROWS_REM)])
        plsc.subcore_barrier()

        def blk(j, _):
            eoff = pl.multiple_of(sid * EPT + j * EB, 8)
            pltpu.sync_copy(dst_hbm.at[pl.ds(eoff, EB)], idxd.at[0])
            pltpu.async_copy(ones_rows, acc.at[idxd.at[0]], ssem, add=True
                             ).wait()
            return 0
        lax.fori_loop(0, NBLK, blk, 0)
        plsc.subcore_barrier()
        pltpu.sync_copy(acc.at[pl.ds(roff, ROWS_PT)],
                        out_hbm.at[pl.ds(roff, ROWS_PT)])

        @pl.when(sid == 0)
        def _rem2():
            pltpu.sync_copy(acc.at[pl.ds(NS * ROWS_PT, ROWS_REM)],
                            out_hbm.at[pl.ds(NS * ROWS_PT, ROWS_REM)])


@functools.lru_cache
def _deg_call():
    return functools.partial(
        pl.kernel,
        out_type=jax.ShapeDtypeStruct((N, 16), jnp.float32),
        mesh=_get_mesh(),
        scratch_types=[
            pltpu.VMEM_SHARED((ACC_ROWS, 16), jnp.float32),
            pltpu.VMEM((ROWS_PT, 16), jnp.float32),
            pltpu.VMEM((EB, 16), jnp.float32),
            pltpu.VMEM((2, EB), jnp.int32),
            pltpu.SemaphoreType.DMA,
        ],
    )(_deg_body)


def _make_agg(K):
    GB = 2

    def body(h, sidx, dst, out, acc, idxs, idxd, rows, gsem, ssem):
        cid = lax.axis_index("c")
        sid = lax.axis_index("s")
        roff = pl.multiple_of(sid * ROWS_PT, 8)
        for p in range(K // NC):
            c = cid * (K // NC) + p
            crow = c * N
            pltpu.sync_copy(h.at[pl.ds(pl.multiple_of(crow + roff, 8),
                                       ROWS_PT)],
                            acc.at[pl.ds(roff, ROWS_PT)])

            @pl.when(sid == 0)
            def _rem():
                pltpu.sync_copy(
                    h.at[pl.ds(pl.multiple_of(crow + NS * ROWS_PT, 8),
                               ROWS_REM)],
                    acc.at[pl.ds(NS * ROWS_PT, ROWS_REM)])
            plsc.subcore_barrier()

            def grp(g, _):
                gets = []
                for b in range(GB):
                    off = pl.multiple_of(sid * EPT + (g * GB + b) * EB, 8)
                    ioff = pl.multiple_of(c * E_PAD + off, 8)
                    pltpu.sync_copy(sidx.at[pl.ds(ioff, EB)], idxs.at[b])
                    pltpu.sync_copy(dst.at[pl.ds(off, EB)], idxd.at[b])
                    gets.append(
                        pltpu.async_copy(h.at[idxs.at[b]], rows.at[b], gsem))
                for hd in gets:
                    hd.wait()
                puts = [
                    pltpu.async_copy(rows.at[b], acc.at[idxd.at[b]], ssem,
                                     add=True)
                    for b in range(GB)
                ]
                for hd in puts:
                    hd.wait()
                return 0
            lax.fori_loop(0, NBLK // GB, grp, 0)
            plsc.subcore_barrier()
            pltpu.sync_copy(acc.at[pl.ds(roff, ROWS_PT)],
                            out.at[pl.ds(pl.multiple_of(crow + roff, 8),
                                         ROWS_PT)])

            @pl.when(sid == 0)
            def _rem2():
                pltpu.sync_copy(
                    acc.at[pl.ds(NS * ROWS_PT, ROWS_REM)],
                    out.at[pl.ds(pl.multiple_of(crow + NS * ROWS_PT, 8),
                                 ROWS_REM)])

    return functools.partial(
        pl.kernel,
        out_type=jax.ShapeDtypeStruct((K * N, 128), jnp.float32),
        mesh=_get_mesh(),
        scratch_types=[
            pltpu.VMEM_SHARED((ACC_ROWS, 128), jnp.float32),
            pltpu.VMEM((GB, EB), jnp.int32),
            pltpu.VMEM((GB, EB), jnp.int32),
            pltpu.VMEM((GB, EB, 128), jnp.float32),
            pltpu.SemaphoreType.DMA,
            pltpu.SemaphoreType.DMA,
        ],
    )(body)


_agg_call = functools.lru_cache(_make_agg)


def _gather_body(z_hbm, idx_hbm, out_hbm, idxv, rowsv, gsem, osem):
    cid = lax.axis_index("c")
    sid = lax.axis_index("s")
    base = (sid * NC + cid) * GPT

    def grp(g, _):
        gets = []
        offs = []
        for b in range(2):
            off = pl.multiple_of(base + (g * 2 + b) * EB, 8)
            offs.append(off)
            pltpu.sync_copy(idx_hbm.at[pl.ds(off, EB)], idxv.at[b])
            gets.append(pltpu.async_copy(z_hbm.at[idxv.at[b]], rowsv.at[b],
                                         gsem))
        for hd in gets:
            hd.wait()
        puts = [
            pltpu.async_copy(rowsv.at[b], out_hbm.at[pl.ds(offs[b], EB)],
                             osem)
            for b in range(2)
        ]
        for hd in puts:
            hd.wait()
        return 0
    lax.fori_loop(0, DBLK // 2, grp, 0)


@functools.lru_cache
def _gather_call():
    return functools.partial(
        pl.kernel,
        out_type=jax.ShapeDtypeStruct((GROWS, 256), jnp.float32),
        mesh=_get_mesh(),
        scratch_types=[
            pltpu.VMEM((2, EB), jnp.int32),
            pltpu.VMEM((2, EB, 256), jnp.float32),
            pltpu.SemaphoreType.DMA,
            pltpu.SemaphoreType.DMA,
        ],
    )(_gather_body)


def _mm_t(x, w):
    return lax.dot_general(x, w, (((1,), (1,)), ((), ())),
                           preferred_element_type=jnp.float32)


def _lrelu(x):
    return jnp.where(x > 0, x, x * 0.01)


def _k1_body(feat, pe31, deg, wpos, bpos, w1, b1, w2, b2, w3, b3, wc1, out):
    x = feat[0]
    pos = _mm_t(pe31[0], wpos[...]) + bpos[...]
    a = _mm_t(x, w1[0, :, :256]) + _mm_t(pos, w1[0, :, 256:]) + b1[0]
    a = _lrelu(a)
    a = _lrelu(_mm_t(a, w2[0]) + b2[0])
    a = _lrelu(_mm_t(a, w3[0]) + b3[0])
    h = _mm_t(a, wc1[...])
    dinv = lax.rsqrt(deg[0][:, 0:1] + 1.0)
    h = h * dinv
    for k in range(4):
        out[k, 0] = h[:, 128 * k:128 * (k + 1)]


def _k1_call(featr, pe31, degr, wpos, bpos, w1, b1, w2, b2, w3, b3, wc1):
    f32 = jnp.float32
    return pl.pallas_call(
        _k1_body,
        grid=(8,),
        in_specs=[
            pl.BlockSpec((1, 1250, 256), lambda i: (i, 0, 0)),
            pl.BlockSpec((1, 1250, 31), lambda i: (i, 0, 0)),
            pl.BlockSpec((1, 1250, 16), lambda i: (i, 0, 0)),
            pl.BlockSpec((8, 31), lambda i: (0, 0)),
            pl.BlockSpec((1, 8), lambda i: (0, 0)),
            pl.BlockSpec((1, 264, 264), lambda i: (i, 0, 0)),
            pl.BlockSpec((1, 1, 264), lambda i: (i, 0, 0)),
            pl.BlockSpec((1, 264, 264), lambda i: (i, 0, 0)),
            pl.BlockSpec((1, 1, 264), lambda i: (i, 0, 0)),
            pl.BlockSpec((1, 264, 264), lambda i: (i, 0, 0)),
            pl.BlockSpec((1, 1, 264), lambda i: (i, 0, 0)),
            pl.BlockSpec((512, 264), lambda i: (0, 0)),
        ],
        out_specs=pl.BlockSpec((4, 1, 1250, 128), lambda i: (0, i, 0, 0)),
        out_shape=jax.ShapeDtypeStruct((4, 8, 1250, 128), f32),
    )(featr, pe31, degr, wpos, bpos, w1, b1, w2, b2, w3, b3, wc1)


def _make_gcn_mid(k_in, k_out):
    def body(agg, deg, bias, w, out):
        x = jnp.concatenate([agg[k] for k in range(k_in)], axis=-1)
        dinv = lax.rsqrt(deg[0][:, 0:1] + 1.0)
        x = jnp.maximum(x * dinv + bias[...], 0.0)
        y = _mm_t(x, w[...]) * dinv
        for k in range(k_out):
            out[k] = y[:, 128 * k:128 * (k + 1)]

    def call(aggr, degr, bias, w):
        d_out = 128 * k_out
        return pl.pallas_call(
            body,
            grid=(10,),
            in_specs=[
                pl.BlockSpec((k_in, 1000, 128), lambda j: (0, j, 0)),
                pl.BlockSpec((1, 1000, 16), lambda j: (j, 0, 0)),
                pl.BlockSpec((1, 128 * k_in), lambda j: (0, 0)),
                pl.BlockSpec((d_out, 128 * k_in), lambda j: (0, 0)),
            ],
            out_specs=pl.BlockSpec((k_out, 1000, 128), lambda j: (0, j, 0)),
            out_shape=jax.ShapeDtypeStruct((k_out, N, 128), jnp.float32),
        )(aggr, degr, bias, w)

    return call


_gcn2_call = _make_gcn_mid(4, 4)
_gcn3_call = _make_gcn_mid(4, 2)


def _z_body(agg, deg, bias, out):
    x = jnp.concatenate([agg[0], agg[1]], axis=-1)
    dinv = lax.rsqrt(deg[0][:, 0:1] + 1.0)
    out[...] = x * dinv + bias[...]


def _z_call(aggr, degr, bias):
    return pl.pallas_call(
        _z_body,
        grid=(10,),
        in_specs=[
            pl.BlockSpec((2, 1000, 128), lambda j: (0, j, 0)),
            pl.BlockSpec((1, 1000, 16), lambda j: (j, 0, 0)),
            pl.BlockSpec((1, 256), lambda j: (0, 0)),
        ],
        out_specs=pl.BlockSpec((1000, 256), lambda j: (j, 0)),
        out_shape=jax.ShapeDtypeStruct((N, 256), jnp.float32),
    )(aggr, degr, bias)


def _dec_body(zi, zj, wd1, bd1, wd2, bd2, out):
    zz = zi[...] * zj[...]
    h = jnp.maximum(_mm_t(zz, wd1[...]) + bd1[...], 0.0)
    y = _mm_t(h, wd2[...]) + bd2[...]
    out[...] = 1.0 / (1.0 + jnp.exp(-y))


def _dec_call(zrows, wd1, bd1, wd2, bd2):
    nb = 98
    return pl.pallas_call(
        _dec_body,
        grid=(nb,),
        in_specs=[
            pl.BlockSpec((1024, 256), lambda j: (j, 0)),
            pl.BlockSpec((1024, 256), lambda j: (j + PAD_PRED // 1024, 0)),
            pl.BlockSpec((256, 256), lambda j: (0, 0)),
            pl.BlockSpec((1, 256), lambda j: (0, 0)),
            pl.BlockSpec((8, 256), lambda j: (0, 0)),
            pl.BlockSpec((1, 8), lambda j: (0, 0)),
        ],
        out_specs=pl.BlockSpec((1024, 8), lambda j: (j, 0)),
        out_shape=jax.ShapeDtypeStruct((nb * 1024, 8), jnp.float32),
    )(zrows, zrows, wd1, bd1, wd2, bd2)


def kernel(adj, features, edges, com_xs, pos_emb, lap_pe, Wpos, bpos,
           W1, b1, W2, b2, W3, b3, Wc1, bc1, Wc2, bc2, Wc3, bc3,
           Wd1, bd1, Wd2, bd2):
    i32 = jnp.int32
    src = adj[0].astype(i32)
    dst = adj[1].astype(i32)

    npad = E_PAD - E
    filler = jnp.arange(npad, dtype=i32)
    src_pad = jnp.concatenate([src, filler % 256])
    dst_pad = jnp.concatenate([dst, N + (filler % NS)])
    srcidx = (src_pad[None, :]
              + (jnp.arange(4, dtype=i32) * N)[:, None]).reshape(-1)

    ppad = PAD_PRED - NPRED
    pfill = jnp.arange(ppad, dtype=i32) % 256
    ei = edges[:, 0].astype(i32)
    ej = edges[:, 1].astype(i32)
    eij = jnp.concatenate([ei, pfill, ej, pfill])

    pe31 = jnp.concatenate([pos_emb, lap_pe], axis=1).reshape(8, 1250, 31)
    featr = features.reshape(8, 1250, 256)

    deg16 = _deg_call()(dst_pad)
    degr8 = deg16.reshape(8, 1250, 16)
    degr10 = deg16.reshape(10, 1000, 16)

    h1p = _k1_call(featr, pe31, degr8, Wpos, bpos.reshape(1, 8),
                   W1, b1.reshape(8, 1, 264), W2, b2.reshape(8, 1, 264),
                   W3, b3.reshape(8, 1, 264), Wc1)
    h1p = h1p.reshape(4 * N, 128)

    agg1 = _agg_call(4)(h1p, srcidx, dst_pad).reshape(4, N, 128)
    h2p = _gcn2_call(agg1, degr10, bc1.reshape(1, 512), Wc2)
    h2p = h2p.reshape(4 * N, 128)

    agg2 = _agg_call(4)(h2p, srcidx, dst_pad).reshape(4, N, 128)
    h3p = _gcn3_call(agg2, degr10, bc2.reshape(1, 512), Wc3)
    h3p = h3p.reshape(2 * N, 128)

    agg3 = _agg_call(2)(h3p, srcidx, dst_pad).reshape(2, N, 128)
    z = _z_call(agg3, degr10, bc3.reshape(1, 256))

    zrows = _gather_call()(z, eij)
    wd2p = jnp.concatenate([Wd2, jnp.zeros((7, 256), jnp.float32)])
    bd2p = jnp.broadcast_to(bd2.reshape(1, 1), (1, 8))
    y = _dec_call(zrows, Wd1, bd1.reshape(1, 256), wd2p, bd2p)
    return y[:NPRED, 0]


# speedup vs baseline: 9.1328x; 1.1969x over previous
import functools

import jax
import jax.numpy as jnp
from jax import lax
from jax.experimental import pallas as pl
from jax.experimental.pallas import tpu as pltpu
from jax.experimental.pallas import tpu_sc as plsc

N = 10000
E = 160000
NPRED = 100000
NC = 2
NS = 16
NT = NC * NS

EB = 128
EPT = 10240
E_PAD = EPT * NS
NBLK = EPT // EB
AEB = 80
ANBLK = EPT // AEB
ARB = 4
ROWS_PT = 624
ROWS_REM = N - NS * ROWS_PT
ACC_ROWS = N + NS
PAD_PRED = 102400
GROWS = 2 * PAD_PRED
GPT = GROWS // NT
DBLK = GPT // EB

@functools.lru_cache
def _get_mesh():
    return plsc.VectorSubcoreMesh(
        core_axis_name="c", subcore_axis_name="s",
        num_cores=NC, num_subcores=NS)


def _deg_body(dst_hbm, out_hbm, acc, zbuf, ones_rows, idxd, ssem):
    cid = lax.axis_index("c")
    sid = lax.axis_index("s")

    @pl.when(cid == 0)
    def _():
        zero16 = jnp.zeros((16,), jnp.float32)
        one16 = jnp.ones((16,), jnp.float32)

        def fill(i, _):
            zbuf[i, :] = zero16
            return 0
        lax.fori_loop(0, ROWS_PT, fill, 0)

        def fill2(i, _):
            ones_rows[i, :] = one16
            return 0
        lax.fori_loop(0, EB, fill2, 0)

        roff = pl.multiple_of(sid * ROWS_PT, 8)
        pltpu.sync_copy(zbuf, acc.at[pl.ds(roff, ROWS_PT)])

        @pl.when(sid == 0)
        def _rem():
            pltpu.sync_copy(zbuf.at[pl.ds(0, ROWS_REM)],
                            acc.at[pl.ds(NS * ROWS_PT, ROWS_REM)])
        plsc.subcore_barrier()

        puts = [None] * NBLK
        for j in range(NBLK):
            b = j % 4
            if j >= 4:
                puts[j - 4].wait()
            eoff = pl.multiple_of(sid * EPT + j * EB, 8)
            pltpu.sync_copy(dst_hbm.at[pl.ds(eoff, EB)], idxd.at[b])
            puts[j] = pltpu.async_copy(ones_rows, acc.at[idxd.at[b]], ssem,
                                       add=True)
        for j in range(NBLK - 4, NBLK):
            puts[j].wait()
        plsc.subcore_barrier()
        pltpu.sync_copy(acc.at[pl.ds(roff, ROWS_PT)],
                        out_hbm.at[pl.ds(roff, ROWS_PT)])

        @pl.when(sid == 0)
        def _rem2():
            pltpu.sync_copy(acc.at[pl.ds(NS * ROWS_PT, ROWS_REM)],
                            out_hbm.at[pl.ds(NS * ROWS_PT, ROWS_REM)])


@functools.lru_cache
def _deg_call():
    return functools.partial(
        pl.kernel,
        out_type=jax.ShapeDtypeStruct((N, 16), jnp.float32),
        mesh=_get_mesh(),
        scratch_types=[
            pltpu.VMEM_SHARED((ACC_ROWS, 16), jnp.float32),
            pltpu.VMEM((ROWS_PT, 16), jnp.float32),
            pltpu.VMEM((EB, 16), jnp.float32),
            pltpu.VMEM((4, EB), jnp.int32),
            pltpu.SemaphoreType.DMA,
        ],
    )(_deg_body)


def _make_agg(K):
    def body(h, sidx, dst, out, acc, idxs, idxd, rows, gsem, ssem):
        cid = lax.axis_index("c")
        sid = lax.axis_index("s")
        roff = pl.multiple_of(sid * ROWS_PT, 8)
        for p in range(K // NC):
            c = cid * (K // NC) + p
            crow = c * N
            pltpu.sync_copy(h.at[pl.ds(pl.multiple_of(crow + roff, 8),
                                       ROWS_PT)],
                            acc.at[pl.ds(roff, ROWS_PT)])

            @pl.when(sid == 0)
            def _rem():
                pltpu.sync_copy(
                    h.at[pl.ds(pl.multiple_of(crow + NS * ROWS_PT, 8),
                               ROWS_REM)],
                    acc.at[pl.ds(NS * ROWS_PT, ROWS_REM)])
            plsc.subcore_barrier()

            gets = [None] * ANBLK
            puts = [None] * ANBLK
            for j in range(ANBLK):
                b = j % ARB
                if j >= ARB:
                    puts[j - ARB].wait()
                off = pl.multiple_of(sid * EPT + j * AEB, 8)
                ioff = pl.multiple_of(c * E_PAD + sid * EPT + j * AEB, 8)
                pltpu.sync_copy(sidx.at[pl.ds(ioff, AEB)], idxs.at[b])
                pltpu.sync_copy(dst.at[pl.ds(off, AEB)], idxd.at[b])
                gets[j] = pltpu.async_copy(h.at[idxs.at[b]], rows.at[b],
                                           gsem)
                if j >= 1:
                    bp = (j - 1) % ARB
                    gets[j - 1].wait()
                    puts[j - 1] = pltpu.async_copy(
                        rows.at[bp], acc.at[idxd.at[bp]], ssem, add=True)
            gets[ANBLK - 1].wait()
            puts[ANBLK - 1] = pltpu.async_copy(
                rows.at[(ANBLK - 1) % ARB],
                acc.at[idxd.at[(ANBLK - 1) % ARB]], ssem, add=True)
            for j in range(ANBLK - ARB, ANBLK):
                puts[j].wait()
            plsc.subcore_barrier()
            pltpu.sync_copy(acc.at[pl.ds(roff, ROWS_PT)],
                            out.at[pl.ds(pl.multiple_of(crow + roff, 8),
                                         ROWS_PT)])

            @pl.when(sid == 0)
            def _rem2():
                pltpu.sync_copy(
                    acc.at[pl.ds(NS * ROWS_PT, ROWS_REM)],
                    out.at[pl.ds(pl.multiple_of(crow + NS * ROWS_PT, 8),
                                 ROWS_REM)])

    return functools.partial(
        pl.kernel,
        out_type=jax.ShapeDtypeStruct((K * N, 128), jnp.float32),
        mesh=_get_mesh(),
        scratch_types=[
            pltpu.VMEM_SHARED((ACC_ROWS, 128), jnp.float32),
            pltpu.VMEM((ARB, AEB), jnp.int32),
            pltpu.VMEM((ARB, AEB), jnp.int32),
            pltpu.VMEM((ARB, AEB, 128), jnp.float32),
            pltpu.SemaphoreType.DMA,
            pltpu.SemaphoreType.DMA,
        ],
    )(body)


_agg_call = functools.lru_cache(_make_agg)


def _gather_body(z_hbm, idx_hbm, out_hbm, idxv, rowsv, gsem, osem):
    cid = lax.axis_index("c")
    sid = lax.axis_index("s")
    base = (sid * NC + cid) * GPT

    gets = [None] * DBLK
    puts = [None] * DBLK
    offs = [None] * DBLK
    for j in range(DBLK):
        b = j % 3
        if j >= 3:
            puts[j - 3].wait()
        offs[j] = pl.multiple_of(base + j * EB, 8)
        pltpu.sync_copy(idx_hbm.at[pl.ds(offs[j], EB)], idxv.at[b])
        gets[j] = pltpu.async_copy(z_hbm.at[idxv.at[b]], rowsv.at[b], gsem)
        if j >= 1:
            bp = (j - 1) % 3
            gets[j - 1].wait()
            puts[j - 1] = pltpu.async_copy(
                rowsv.at[bp], out_hbm.at[pl.ds(offs[j - 1], EB)], osem)
    gets[DBLK - 1].wait()
    puts[DBLK - 1] = pltpu.async_copy(
        rowsv.at[(DBLK - 1) % 3], out_hbm.at[pl.ds(offs[DBLK - 1], EB)],
        osem)
    for j in range(DBLK - 3, DBLK):
        puts[j].wait()


@functools.lru_cache
def _gather_call():
    return functools.partial(
        pl.kernel,
        out_type=jax.ShapeDtypeStruct((GROWS, 256), jnp.float32),
        mesh=_get_mesh(),
        scratch_types=[
            pltpu.VMEM((3, EB), jnp.int32),
            pltpu.VMEM((3, EB, 256), jnp.float32),
            pltpu.SemaphoreType.DMA,
            pltpu.SemaphoreType.DMA,
        ],
    )(_gather_body)


def _mm_t(x, w):
    return lax.dot_general(x, w, (((1,), (1,)), ((), ())),
                           preferred_element_type=jnp.float32)


def _lrelu(x):
    return jnp.where(x > 0, x, x * 0.01)


def _k1_body(feat, pe31, deg, wpos, bpos, w1, b1, w2, b2, w3, b3, wc1, out):
    x = feat[0]
    pos = _mm_t(pe31[0], wpos[...]) + bpos[...]
    a = _mm_t(x, w1[0, :, :256]) + _mm_t(pos, w1[0, :, 256:]) + b1[0]
    a = _lrelu(a)
    a = _lrelu(_mm_t(a, w2[0]) + b2[0])
    a = _lrelu(_mm_t(a, w3[0]) + b3[0])
    h = _mm_t(a, wc1[...])
    dinv = lax.rsqrt(deg[0][:, 0:1] + 1.0)
    h = h * dinv
    for k in range(4):
        out[k, 0] = h[:, 128 * k:128 * (k + 1)]


def _k1_call(featr, pe31, degr, wpos, bpos, w1, b1, w2, b2, w3, b3, wc1):
    f32 = jnp.float32
    return pl.pallas_call(
        _k1_body,
        grid=(8,),
        in_specs=[
            pl.BlockSpec((1, 1250, 256), lambda i: (i, 0, 0)),
            pl.BlockSpec((1, 1250, 31), lambda i: (i, 0, 0)),
            pl.BlockSpec((1, 1250, 16), lambda i: (i, 0, 0)),
            pl.BlockSpec((8, 31), lambda i: (0, 0)),
            pl.BlockSpec((1, 8), lambda i: (0, 0)),
            pl.BlockSpec((1, 264, 264), lambda i: (i, 0, 0)),
            pl.BlockSpec((1, 1, 264), lambda i: (i, 0, 0)),
            pl.BlockSpec((1, 264, 264), lambda i: (i, 0, 0)),
            pl.BlockSpec((1, 1, 264), lambda i: (i, 0, 0)),
            pl.BlockSpec((1, 264, 264), lambda i: (i, 0, 0)),
            pl.BlockSpec((1, 1, 264), lambda i: (i, 0, 0)),
            pl.BlockSpec((512, 264), lambda i: (0, 0)),
        ],
        out_specs=pl.BlockSpec((4, 1, 1250, 128), lambda i: (0, i, 0, 0)),
        out_shape=jax.ShapeDtypeStruct((4, 8, 1250, 128), f32),
    )(featr, pe31, degr, wpos, bpos, w1, b1, w2, b2, w3, b3, wc1)


def _make_gcn_mid(k_in, k_out):
    def body(agg, deg, bias, w, out):
        x = jnp.concatenate([agg[k] for k in range(k_in)], axis=-1)
        dinv = lax.rsqrt(deg[0][:, 0:1] + 1.0)
        x = jnp.maximum(x * dinv + bias[...], 0.0)
        y = _mm_t(x, w[...]) * dinv
        for k in range(k_out):
            out[k] = y[:, 128 * k:128 * (k + 1)]

    def call(aggr, degr, bias, w):
        d_out = 128 * k_out
        return pl.pallas_call(
            body,
            grid=(10,),
            in_specs=[
                pl.BlockSpec((k_in, 1000, 128), lambda j: (0, j, 0)),
                pl.BlockSpec((1, 1000, 16), lambda j: (j, 0, 0)),
                pl.BlockSpec((1, 128 * k_in), lambda j: (0, 0)),
                pl.BlockSpec((d_out, 128 * k_in), lambda j: (0, 0)),
            ],
            out_specs=pl.BlockSpec((k_out, 1000, 128), lambda j: (0, j, 0)),
            out_shape=jax.ShapeDtypeStruct((k_out, N, 128), jnp.float32),
        )(aggr, degr, bias, w)

    return call


_gcn2_call = _make_gcn_mid(4, 4)
_gcn3_call = _make_gcn_mid(4, 2)


def _z_body(agg, deg, bias, out):
    x = jnp.concatenate([agg[0], agg[1]], axis=-1)
    dinv = lax.rsqrt(deg[0][:, 0:1] + 1.0)
    out[...] = x * dinv + bias[...]


def _z_call(aggr, degr, bias):
    return pl.pallas_call(
        _z_body,
        grid=(10,),
        in_specs=[
            pl.BlockSpec((2, 1000, 128), lambda j: (0, j, 0)),
            pl.BlockSpec((1, 1000, 16), lambda j: (j, 0, 0)),
            pl.BlockSpec((1, 256), lambda j: (0, 0)),
        ],
        out_specs=pl.BlockSpec((1000, 256), lambda j: (j, 0)),
        out_shape=jax.ShapeDtypeStruct((N, 256), jnp.float32),
    )(aggr, degr, bias)


def _dec_body(zi, zj, wd1, bd1, wd2, bd2, out):
    zz = zi[...] * zj[...]
    h = jnp.maximum(_mm_t(zz, wd1[...]) + bd1[...], 0.0)
    y = _mm_t(h, wd2[...]) + bd2[...]
    out[...] = 1.0 / (1.0 + jnp.exp(-y))


def _dec_call(zrows, wd1, bd1, wd2, bd2):
    nb = 98
    return pl.pallas_call(
        _dec_body,
        grid=(nb,),
        in_specs=[
            pl.BlockSpec((1024, 256), lambda j: (j, 0)),
            pl.BlockSpec((1024, 256), lambda j: (j + PAD_PRED // 1024, 0)),
            pl.BlockSpec((256, 256), lambda j: (0, 0)),
            pl.BlockSpec((1, 256), lambda j: (0, 0)),
            pl.BlockSpec((8, 256), lambda j: (0, 0)),
            pl.BlockSpec((1, 8), lambda j: (0, 0)),
        ],
        out_specs=pl.BlockSpec((1024, 8), lambda j: (j, 0)),
        out_shape=jax.ShapeDtypeStruct((nb * 1024, 8), jnp.float32),
    )(zrows, zrows, wd1, bd1, wd2, bd2)


def kernel(adj, features, edges, com_xs, pos_emb, lap_pe, Wpos, bpos,
           W1, b1, W2, b2, W3, b3, Wc1, bc1, Wc2, bc2, Wc3, bc3,
           Wd1, bd1, Wd2, bd2):
    i32 = jnp.int32
    src = adj[0].astype(i32)
    dst = adj[1].astype(i32)

    npad = E_PAD - E
    filler = jnp.arange(npad, dtype=i32)
    src_pad = jnp.concatenate([src, filler % 256])
    dst_pad = jnp.concatenate([dst, N + (filler % NS)])
    srcidx = (src_pad[None, :]
              + (jnp.arange(4, dtype=i32) * N)[:, None]).reshape(-1)

    ppad = PAD_PRED - NPRED
    pfill = jnp.arange(ppad, dtype=i32) % 256
    ei = edges[:, 0].astype(i32)
    ej = edges[:, 1].astype(i32)
    eij = jnp.concatenate([ei, pfill, ej, pfill])

    pe31 = jnp.concatenate([pos_emb, lap_pe], axis=1).reshape(8, 1250, 31)
    featr = features.reshape(8, 1250, 256)

    deg16 = _deg_call()(dst_pad)
    degr8 = deg16.reshape(8, 1250, 16)
    degr10 = deg16.reshape(10, 1000, 16)

    h1p = _k1_call(featr, pe31, degr8, Wpos, bpos.reshape(1, 8),
                   W1, b1.reshape(8, 1, 264), W2, b2.reshape(8, 1, 264),
                   W3, b3.reshape(8, 1, 264), Wc1)
    h1p = h1p.reshape(4 * N, 128)

    agg1 = _agg_call(4)(h1p, srcidx, dst_pad).reshape(4, N, 128)
    h2p = _gcn2_call(agg1, degr10, bc1.reshape(1, 512), Wc2)
    h2p = h2p.reshape(4 * N, 128)

    agg2 = _agg_call(4)(h2p, srcidx, dst_pad).reshape(4, N, 128)
    h3p = _gcn3_call(agg2, degr10, bc2.reshape(1, 512), Wc3)
    h3p = h3p.reshape(2 * N, 128)

    agg3 = _agg_call(2)(h3p, srcidx, dst_pad).reshape(2, N, 128)
    z = _z_call(agg3, degr10, bc3.reshape(1, 256))

    zrows = _gather_call()(z, eij)
    wd2p = jnp.concatenate([Wd2, jnp.zeros((7, 256), jnp.float32)])
    bd2p = jnp.broadcast_to(bd2.reshape(1, 1), (1, 8))
    y = _dec_call(zrows, Wd1, bd1.reshape(1, 256), wd2p, bd2p)
    return y[:NPRED, 0]


# speedup vs baseline: 12.1790x; 1.3335x over previous
import functools

import jax
import jax.numpy as jnp
from jax import lax
from jax.experimental import pallas as pl
from jax.experimental.pallas import tpu as pltpu
from jax.experimental.pallas import tpu_sc as plsc

N = 10000
E = 160000
NPRED = 100000
NC = 2
NS = 16
NT = NC * NS

EB = 128
EPT = 10240
E_PAD = EPT * NS
NBLK = EPT // EB
AEB = 80
ANBLK = EPT // AEB
ARB = 4
ROWS_PT = 624
ROWS_REM = N - NS * ROWS_PT
ACC_ROWS = N + NS
PAD_PRED = 102400
GROWS = 2 * PAD_PRED
GPT = GROWS // NT
DBLK = GPT // EB

@functools.lru_cache
def _get_mesh():
    return plsc.VectorSubcoreMesh(
        core_axis_name="c", subcore_axis_name="s",
        num_cores=NC, num_subcores=NS)


def _deg_body(dst_hbm, out_hbm, acc, zbuf, ones_rows, idxd, ssem, isem):
    cid = lax.axis_index("c")
    sid = lax.axis_index("s")

    @pl.when(cid == 0)
    def _():
        zero16 = jnp.zeros((16,), jnp.float32)
        one16 = jnp.ones((16,), jnp.float32)

        def fill(i, _):
            zbuf[i, :] = zero16
            return 0
        lax.fori_loop(0, ROWS_PT, fill, 0)

        def fill2(i, _):
            ones_rows[i, :] = one16
            return 0
        lax.fori_loop(0, EB, fill2, 0)

        roff = pl.multiple_of(sid * ROWS_PT, 8)
        pltpu.sync_copy(zbuf, acc.at[pl.ds(roff, ROWS_PT)])

        @pl.when(sid == 0)
        def _rem():
            pltpu.sync_copy(zbuf.at[pl.ds(0, ROWS_REM)],
                            acc.at[pl.ds(NS * ROWS_PT, ROWS_REM)])
        plsc.subcore_barrier()

        puts = [None] * NBLK
        ixh = [None] * NBLK

        def fire_idx(j):
            eoff = pl.multiple_of(sid * EPT + j * EB, 8)
            ixh[j] = pltpu.async_copy(dst_hbm.at[pl.ds(eoff, EB)],
                                      idxd.at[j % 8], isem)

        fire_idx(0)
        fire_idx(1)
        for j in range(NBLK):
            if j >= 4:
                puts[j - 4].wait()
            if j + 2 < NBLK:
                fire_idx(j + 2)
            ixh[j].wait()
            puts[j] = pltpu.async_copy(ones_rows, acc.at[idxd.at[j % 8]],
                                       ssem, add=True)
        for j in range(NBLK - 4, NBLK):
            puts[j].wait()
        plsc.subcore_barrier()
        pltpu.sync_copy(acc.at[pl.ds(roff, ROWS_PT)],
                        out_hbm.at[pl.ds(roff, ROWS_PT)])

        @pl.when(sid == 0)
        def _rem2():
            pltpu.sync_copy(acc.at[pl.ds(NS * ROWS_PT, ROWS_REM)],
                            out_hbm.at[pl.ds(NS * ROWS_PT, ROWS_REM)])


@functools.lru_cache
def _deg_call():
    return functools.partial(
        pl.kernel,
        out_type=jax.ShapeDtypeStruct((N, 16), jnp.float32),
        mesh=_get_mesh(),
        scratch_types=[
            pltpu.VMEM_SHARED((ACC_ROWS, 16), jnp.float32),
            pltpu.VMEM((ROWS_PT, 16), jnp.float32),
            pltpu.VMEM((EB, 16), jnp.float32),
            pltpu.VMEM((8, EB), jnp.int32),
            pltpu.SemaphoreType.DMA,
            pltpu.SemaphoreType.DMA,
        ],
    )(_deg_body)


def _make_agg(K):
    def body(h, sidx, dst, out, acc, idxs, idxd, rows, gsem, ssem, isem):
        cid = lax.axis_index("c")
        sid = lax.axis_index("s")
        roff = pl.multiple_of(sid * ROWS_PT, 8)
        for p in range(K // NC):
            c = cid * (K // NC) + p
            crow = c * N
            pltpu.sync_copy(h.at[pl.ds(pl.multiple_of(crow + roff, 8),
                                       ROWS_PT)],
                            acc.at[pl.ds(roff, ROWS_PT)])

            @pl.when(sid == 0)
            def _rem():
                pltpu.sync_copy(
                    h.at[pl.ds(pl.multiple_of(crow + NS * ROWS_PT, 8),
                               ROWS_REM)],
                    acc.at[pl.ds(NS * ROWS_PT, ROWS_REM)])
            plsc.subcore_barrier()

            gets = [None] * ANBLK
            puts = [None] * ANBLK
            ixh = [None] * ANBLK

            def fire_idx(j):
                s = j % 8
                off = pl.multiple_of(sid * EPT + j * AEB, 8)
                ioff = pl.multiple_of(c * E_PAD + sid * EPT + j * AEB, 8)
                ixh[j] = (
                    pltpu.async_copy(sidx.at[pl.ds(ioff, AEB)], idxs.at[s],
                                     isem),
                    pltpu.async_copy(dst.at[pl.ds(off, AEB)], idxd.at[s],
                                     isem))

            def fire_put(j):
                puts[j] = pltpu.async_copy(
                    rows.at[j % ARB], acc.at[idxd.at[j % 8]], ssem, add=True)

            fire_idx(0)
            fire_idx(1)
            for j in range(ANBLK):
                if j >= ARB:
                    puts[j - ARB].wait()
                if j + 2 < ANBLK:
                    fire_idx(j + 2)
                ixh[j][0].wait()
                ixh[j][1].wait()
                gets[j] = pltpu.async_copy(h.at[idxs.at[j % 8]],
                                           rows.at[j % ARB], gsem)
                if j >= 2:
                    gets[j - 2].wait()
                    fire_put(j - 2)
            for j in range(ANBLK - 2, ANBLK):
                gets[j].wait()
                fire_put(j)
            for j in range(ANBLK - ARB, ANBLK):
                puts[j].wait()
            plsc.subcore_barrier()
            pltpu.sync_copy(acc.at[pl.ds(roff, ROWS_PT)],
                            out.at[pl.ds(pl.multiple_of(crow + roff, 8),
                                         ROWS_PT)])

            @pl.when(sid == 0)
            def _rem2():
                pltpu.sync_copy(
                    acc.at[pl.ds(NS * ROWS_PT, ROWS_REM)],
                    out.at[pl.ds(pl.multiple_of(crow + NS * ROWS_PT, 8),
                                 ROWS_REM)])

    return functools.partial(
        pl.kernel,
        out_type=jax.ShapeDtypeStruct((K * N, 128), jnp.float32),
        mesh=_get_mesh(),
        scratch_types=[
            pltpu.VMEM_SHARED((ACC_ROWS, 128), jnp.float32),
            pltpu.VMEM((8, AEB), jnp.int32),
            pltpu.VMEM((8, AEB), jnp.int32),
            pltpu.VMEM((ARB, AEB, 128), jnp.float32),
            pltpu.SemaphoreType.DMA,
            pltpu.SemaphoreType.DMA,
            pltpu.SemaphoreType.DMA,
        ],
    )(body)


_agg_call = functools.lru_cache(_make_agg)


def _gather_body(z_hbm, idx_hbm, out_hbm, idxv, rowsv, gsem, osem, isem):
    cid = lax.axis_index("c")
    sid = lax.axis_index("s")
    base = (sid * NC + cid) * GPT

    gets = [None] * DBLK
    puts = [None] * DBLK
    offs = [None] * DBLK
    ixh = [None] * DBLK

    def fire_idx(j):
        offs[j] = pl.multiple_of(base + j * EB, 8)
        ixh[j] = pltpu.async_copy(idx_hbm.at[pl.ds(offs[j], EB)],
                                  idxv.at[j % 8], isem)

    fire_idx(0)
    fire_idx(1)
    for j in range(DBLK):
        if j >= 3:
            puts[j - 3].wait()
        if j + 2 < DBLK:
            fire_idx(j + 2)
        ixh[j].wait()
        gets[j] = pltpu.async_copy(z_hbm.at[idxv.at[j % 8]], rowsv.at[j % 3],
                                   gsem)
        if j >= 1:
            gets[j - 1].wait()
            puts[j - 1] = pltpu.async_copy(
                rowsv.at[(j - 1) % 3], out_hbm.at[pl.ds(offs[j - 1], EB)],
                osem)
    gets[DBLK - 1].wait()
    puts[DBLK - 1] = pltpu.async_copy(
        rowsv.at[(DBLK - 1) % 3], out_hbm.at[pl.ds(offs[DBLK - 1], EB)],
        osem)
    for j in range(DBLK - 3, DBLK):
        puts[j].wait()


@functools.lru_cache
def _gather_call():
    return functools.partial(
        pl.kernel,
        out_type=jax.ShapeDtypeStruct((GROWS, 256), jnp.float32),
        mesh=_get_mesh(),
        scratch_types=[
            pltpu.VMEM((8, EB), jnp.int32),
            pltpu.VMEM((3, EB, 256), jnp.float32),
            pltpu.SemaphoreType.DMA,
            pltpu.SemaphoreType.DMA,
            pltpu.SemaphoreType.DMA,
        ],
    )(_gather_body)


def _mm_t(x, w):
    return lax.dot_general(x, w, (((1,), (1,)), ((), ())),
                           preferred_element_type=jnp.float32)


def _lrelu(x):
    return jnp.where(x > 0, x, x * 0.01)


def _k1_body(feat, pe31, deg, wpos, bpos, w1, b1, w2, b2, w3, b3, wc1, out):
    x = feat[0]
    pos = _mm_t(pe31[0], wpos[...]) + bpos[...]
    a = _mm_t(x, w1[0, :, :256]) + _mm_t(pos, w1[0, :, 256:]) + b1[0]
    a = _lrelu(a)
    a = _lrelu(_mm_t(a, w2[0]) + b2[0])
    a = _lrelu(_mm_t(a, w3[0]) + b3[0])
    h = _mm_t(a, wc1[...])
    dinv = lax.rsqrt(deg[0][:, 0:1] + 1.0)
    h = h * dinv
    for k in range(4):
        out[k, 0] = h[:, 128 * k:128 * (k + 1)]


def _k1_call(featr, pe31, degr, wpos, bpos, w1, b1, w2, b2, w3, b3, wc1):
    f32 = jnp.float32
    return pl.pallas_call(
        _k1_body,
        grid=(8,),
        in_specs=[
            pl.BlockSpec((1, 1250, 256), lambda i: (i, 0, 0)),
            pl.BlockSpec((1, 1250, 31), lambda i: (i, 0, 0)),
            pl.BlockSpec((1, 1250, 16), lambda i: (i, 0, 0)),
            pl.BlockSpec((8, 31), lambda i: (0, 0)),
            pl.BlockSpec((1, 8), lambda i: (0, 0)),
            pl.BlockSpec((1, 264, 264), lambda i: (i, 0, 0)),
            pl.BlockSpec((1, 1, 264), lambda i: (i, 0, 0)),
            pl.BlockSpec((1, 264, 264), lambda i: (i, 0, 0)),
            pl.BlockSpec((1, 1, 264), lambda i: (i, 0, 0)),
            pl.BlockSpec((1, 264, 264), lambda i: (i, 0, 0)),
            pl.BlockSpec((1, 1, 264), lambda i: (i, 0, 0)),
            pl.BlockSpec((512, 264), lambda i: (0, 0)),
        ],
        out_specs=pl.BlockSpec((4, 1, 1250, 128), lambda i: (0, i, 0, 0)),
        out_shape=jax.ShapeDtypeStruct((4, 8, 1250, 128), f32),
    )(featr, pe31, degr, wpos, bpos, w1, b1, w2, b2, w3, b3, wc1)


def _make_gcn_mid(k_in, k_out):
    def body(agg, deg, bias, w, out):
        x = jnp.concatenate([agg[k] for k in range(k_in)], axis=-1)
        dinv = lax.rsqrt(deg[0][:, 0:1] + 1.0)
        x = jnp.maximum(x * dinv + bias[...], 0.0)
        y = _mm_t(x, w[...]) * dinv
        for k in range(k_out):
            out[k] = y[:, 128 * k:128 * (k + 1)]

    def call(aggr, degr, bias, w):
        d_out = 128 * k_out
        return pl.pallas_call(
            body,
            grid=(10,),
            in_specs=[
                pl.BlockSpec((k_in, 1000, 128), lambda j: (0, j, 0)),
                pl.BlockSpec((1, 1000, 16), lambda j: (j, 0, 0)),
                pl.BlockSpec((1, 128 * k_in), lambda j: (0, 0)),
                pl.BlockSpec((d_out, 128 * k_in), lambda j: (0, 0)),
            ],
            out_specs=pl.BlockSpec((k_out, 1000, 128), lambda j: (0, j, 0)),
            out_shape=jax.ShapeDtypeStruct((k_out, N, 128), jnp.float32),
        )(aggr, degr, bias, w)

    return call


_gcn2_call = _make_gcn_mid(4, 4)
_gcn3_call = _make_gcn_mid(4, 2)


def _z_body(agg, deg, bias, out):
    x = jnp.concatenate([agg[0], agg[1]], axis=-1)
    dinv = lax.rsqrt(deg[0][:, 0:1] + 1.0)
    out[...] = x * dinv + bias[...]


def _z_call(aggr, degr, bias):
    return pl.pallas_call(
        _z_body,
        grid=(10,),
        in_specs=[
            pl.BlockSpec((2, 1000, 128), lambda j: (0, j, 0)),
            pl.BlockSpec((1, 1000, 16), lambda j: (j, 0, 0)),
            pl.BlockSpec((1, 256), lambda j: (0, 0)),
        ],
        out_specs=pl.BlockSpec((1000, 256), lambda j: (j, 0)),
        out_shape=jax.ShapeDtypeStruct((N, 256), jnp.float32),
    )(aggr, degr, bias)


def _dec_body(zi, zj, wd1, bd1, wd2, bd2, out):
    zz = zi[...] * zj[...]
    h = jnp.maximum(_mm_t(zz, wd1[...]) + bd1[...], 0.0)
    y = _mm_t(h, wd2[...]) + bd2[...]
    out[...] = 1.0 / (1.0 + jnp.exp(-y))


def _dec_call(zrows, wd1, bd1, wd2, bd2):
    nb = 98
    return pl.pallas_call(
        _dec_body,
        grid=(nb,),
        in_specs=[
            pl.BlockSpec((1024, 256), lambda j: (j, 0)),
            pl.BlockSpec((1024, 256), lambda j: (j + PAD_PRED // 1024, 0)),
            pl.BlockSpec((256, 256), lambda j: (0, 0)),
            pl.BlockSpec((1, 256), lambda j: (0, 0)),
            pl.BlockSpec((8, 256), lambda j: (0, 0)),
            pl.BlockSpec((1, 8), lambda j: (0, 0)),
        ],
        out_specs=pl.BlockSpec((1024, 8), lambda j: (j, 0)),
        out_shape=jax.ShapeDtypeStruct((nb * 1024, 8), jnp.float32),
    )(zrows, zrows, wd1, bd1, wd2, bd2)


def kernel(adj, features, edges, com_xs, pos_emb, lap_pe, Wpos, bpos,
           W1, b1, W2, b2, W3, b3, Wc1, bc1, Wc2, bc2, Wc3, bc3,
           Wd1, bd1, Wd2, bd2):
    i32 = jnp.int32
    src = adj[0].astype(i32)
    dst = adj[1].astype(i32)

    npad = E_PAD - E
    filler = jnp.arange(npad, dtype=i32)
    src_pad = jnp.concatenate([src, filler % 256])
    dst_pad = jnp.concatenate([dst, N + (filler % NS)])
    srcidx = (src_pad[None, :]
              + (jnp.arange(4, dtype=i32) * N)[:, None]).reshape(-1)

    ppad = PAD_PRED - NPRED
    pfill = jnp.arange(ppad, dtype=i32) % 256
    ei = edges[:, 0].astype(i32)
    ej = edges[:, 1].astype(i32)
    eij = jnp.concatenate([ei, pfill, ej, pfill])

    pe31 = jnp.concatenate([pos_emb, lap_pe], axis=1).reshape(8, 1250, 31)
    featr = features.reshape(8, 1250, 256)

    deg16 = _deg_call()(dst_pad)
    degr8 = deg16.reshape(8, 1250, 16)
    degr10 = deg16.reshape(10, 1000, 16)

    h1p = _k1_call(featr, pe31, degr8, Wpos, bpos.reshape(1, 8),
                   W1, b1.reshape(8, 1, 264), W2, b2.reshape(8, 1, 264),
                   W3, b3.reshape(8, 1, 264), Wc1)
    h1p = h1p.reshape(4 * N, 128)

    agg1 = _agg_call(4)(h1p, srcidx, dst_pad).reshape(4, N, 128)
    h2p = _gcn2_call(agg1, degr10, bc1.reshape(1, 512), Wc2)
    h2p = h2p.reshape(4 * N, 128)

    agg2 = _agg_call(4)(h2p, srcidx, dst_pad).reshape(4, N, 128)
    h3p = _gcn3_call(agg2, degr10, bc2.reshape(1, 512), Wc3)
    h3p = h3p.reshape(2 * N, 128)

    agg3 = _agg_call(2)(h3p, srcidx, dst_pad).reshape(2, N, 128)
    z = _z_call(agg3, degr10, bc3.reshape(1, 256))

    zrows = _gather_call()(z, eij)
    wd2p = jnp.concatenate([Wd2, jnp.zeros((7, 256), jnp.float32)])
    bd2p = jnp.broadcast_to(bd2.reshape(1, 1), (1, 8))
    y = _dec_call(zrows, Wd1, bd1.reshape(1, 256), wd2p, bd2p)
    return y[:NPRED, 0]
